# baseline (speedup 1.0000x reference)
import functools

import jax
import jax.numpy as jnp
from jax import lax
from jax.experimental import pallas as pl
from jax.experimental.pallas import tpu as pltpu
from jax.experimental.pallas import tpu_sc as plsc

N = 10000
E = 320000
HID = 16
NPAD = 10240
NW = 32
W = 128
K = 80
EPAD = NW * K * W
GROUP = 8
NG = K // GROUP
KD = 2 * K
NGD = KD // GROUP
ROWS_PER_SUB = NPAD // 16

_mesh = plsc.VectorSubcoreMesh(core_axis_name="c", subcore_axis_name="s")


def _newton_rsqrt(x):
    i = lax.bitcast_convert_type(x, jnp.int32)
    i = jnp.int32(0x5F3759DF) - lax.shift_right_arithmetic(i, 1)
    y = lax.bitcast_convert_type(i, jnp.float32)
    for _ in range(4):
        y = y * (1.5 - 0.5 * x * y * y)
    return y


def _tc1_body(x_ref, w1_ref, h_ref):
    h = jnp.dot(x_ref[...], w1_ref[...], preferred_element_type=jnp.float32)
    h_ref[0:N, :] = h
    h_ref[N:NPAD, :] = jnp.zeros((NPAD - N, HID), jnp.float32)


def _tc1(x, w1):
    return pl.pallas_call(
        _tc1_body,
        out_shape=jax.ShapeDtypeStruct((NPAD, HID), jnp.float32),
    )(x, w1)


@functools.partial(
    pl.kernel,
    out_type=(
        jax.ShapeDtypeStruct((2 * NPAD, HID), jnp.float32),
        jax.ShapeDtypeStruct((2 * NPAD,), jnp.float32),
        jax.ShapeDtypeStruct((NPAD, HID), jnp.float32),
        jax.ShapeDtypeStruct((NPAD,), jnp.float32),
    ),
    mesh=_mesh,
    scratch_types=[
        pltpu.VMEM((KD, W), jnp.int32),
        pltpu.VMEM((KD, W), jnp.int32),
        pltpu.VMEM((W,), jnp.float32),
        pltpu.VMEM((2, GROUP * W), jnp.float32),
        pltpu.VMEM((2, GROUP * W, HID), jnp.float32),
        pltpu.VMEM((ROWS_PER_SUB, HID), jnp.float32),
        pltpu.VMEM((ROWS_PER_SUB,), jnp.float32),
        pltpu.VMEM_SHARED((NPAD, HID), jnp.float32),
        pltpu.VMEM_SHARED((NPAD,), jnp.float32),
        pltpu.VMEM_SHARED((NPAD,), jnp.float32),
        pltpu.SemaphoreType.DMA,
        pltpu.SemaphoreType.DMA,
        pltpu.SemaphoreType.DMA,
        pltpu.SemaphoreType.DMA,
    ],
    compiler_params=pltpu.CompilerParams(use_tc_tiling_on_sc=False),
)
def _main_kernel(dstd_hbm, srcd_hbm, h_hbm,
                 acc_out, cpre_out, hp_out, dis_out,
                 dstd_v, srcd_v, ones_v, vals_v, rows_v,
                 stage_v, zbuf,
                 acc_sh, cpre_sh, dis_sh, sem_gr, sem_gv, sem_sr, sem_sv):
    c = lax.axis_index("c")
    s = lax.axis_index("s")
    base = s * ROWS_PER_SUB
    cK = c * K

    def zrow(i, carry):
        stage_v[i, :] = jnp.zeros((16,), jnp.float32)
        return carry

    lax.fori_loop(0, ROWS_PER_SUB, zrow, 0)

    def zsca(i, carry):
        zbuf[pl.ds(i * 16, 16)] = jnp.zeros((16,), jnp.float32)
        return carry

    lax.fori_loop(0, ROWS_PER_SUB // 16, zsca, 0)
    for o in range(W // 16):
        ones_v[pl.ds(o * 16, 16)] = jnp.ones((16,), jnp.float32)
    pltpu.sync_copy(stage_v, acc_sh.at[pl.ds(base, ROWS_PER_SUB)])
    pltpu.sync_copy(zbuf, cpre_sh.at[pl.ds(base, ROWS_PER_SUB)])
    pltpu.sync_copy(zbuf, dis_sh.at[pl.ds(base, ROWS_PER_SUB)])
    pltpu.sync_copy(dstd_hbm.at[s], dstd_v)
    pltpu.sync_copy(srcd_hbm.at[s], srcd_v)
    plsc.subcore_barrier()

    def dfire(g):
        for k in range(GROUP):
            pltpu.async_copy(ones_v, dis_sh.at[dstd_v.at[g * GROUP + k]],
                             sem_gr, add=True)

    def ddrain(g):
        for k in range(GROUP):
            pltpu.make_async_copy(ones_v,
                                  dis_sh.at[dstd_v.at[g * GROUP + k]],
                                  sem_gr).wait()

    dfire(0)

    def dbody(g, carry):
        @pl.when(g + 1 < NGD)
        def _():
            dfire(g + 1)

        ddrain(g)
        return carry

    lax.fori_loop(0, NGD, dbody, 0)
    plsc.subcore_barrier()

    pltpu.sync_copy(dis_sh.at[pl.ds(base, ROWS_PER_SUB)], zbuf)
    pltpu.sync_copy(h_hbm.at[pl.ds(base, ROWS_PER_SUB)], stage_v)

    def dis_vec(i, carry):
        d = zbuf[pl.ds(i * 16, 16)] + 1.0
        zbuf[pl.ds(i * 16, 16)] = _newton_rsqrt(d)
        return carry

    lax.fori_loop(0, ROWS_PER_SUB // 16, dis_vec, 0)

    def scale_blk(i, carry):
        dv = zbuf[pl.ds(i * 16, 16)]
        for k in range(16):
            r = i * 16 + k
            stage_v[r, :] = stage_v[r, :] * dv[k]
        return carry

    lax.fori_loop(0, ROWS_PER_SUB // 16, scale_blk, 0)
    pltpu.sync_copy(zbuf, dis_sh.at[pl.ds(base, ROWS_PER_SUB)])
    pltpu.sync_copy(zbuf, dis_out.at[pl.ds(base, ROWS_PER_SUB)])
    pltpu.sync_copy(stage_v, hp_out.at[pl.ds(base, ROWS_PER_SUB)])
    plsc.subcore_barrier()

    def fire_gathers(g, b):
        for k in range(GROUP):
            j = cK + g * GROUP + k
            pltpu.async_copy(hp_out.at[srcd_v.at[j]],
                             rows_v.at[b, pl.ds(k * W, W)], sem_gr)
            pltpu.async_copy(dis_sh.at[dstd_v.at[j]],
                             vals_v.at[b, pl.ds(k * W, W)], sem_gv)

    def drain_gathers(g, b):
        for k in range(GROUP):
            j = cK + g * GROUP + k
            pltpu.make_async_copy(hp_out.at[srcd_v.at[j]],
                                  rows_v.at[b, pl.ds(k * W, W)], sem_gr).wait()
            pltpu.make_async_copy(dis_sh.at[dstd_v.at[j]],
                                  vals_v.at[b, pl.ds(k * W, W)], sem_gv).wait()

    def fire_scatters(g, b):
        for k in range(GROUP):
            j = cK + g * GROUP + k
            pltpu.async_copy(rows_v.at[b, pl.ds(k * W, W)],
                             acc_sh.at[dstd_v.at[j]], sem_sr, add=True)
            pltpu.async_copy(vals_v.at[b, pl.ds(k * W, W)],
                             cpre_sh.at[srcd_v.at[j]], sem_sv, add=True)

    def drain_scatters(g, b):
        for k in range(GROUP):
            j = cK + g * GROUP + k
            pltpu.make_async_copy(rows_v.at[b, pl.ds(k * W, W)],
                                  acc_sh.at[dstd_v.at[j]], sem_sr).wait()
            pltpu.make_async_copy(vals_v.at[b, pl.ds(k * W, W)],
                                  cpre_sh.at[srcd_v.at[j]], sem_sv).wait()

    fire_gathers(0, 0)

    def body(g, carry):
        b = lax.rem(g, 2)
        drain_gathers(g, b)

        @pl.when(g + 1 < NG)
        def _():
            fire_gathers(g + 1, 1 - b)

        fire_scatters(g, b)
        drain_scatters(g, b)
        return carry

    lax.fori_loop(0, NG, body, 0)
    plsc.subcore_barrier()
    pltpu.sync_copy(acc_sh.at[pl.ds(base, ROWS_PER_SUB)], stage_v)
    pltpu.sync_copy(stage_v, acc_out.at[pl.ds(c * NPAD + base, ROWS_PER_SUB)])
    pltpu.sync_copy(cpre_sh.at[pl.ds(base, ROWS_PER_SUB)], zbuf)
    pltpu.sync_copy(zbuf, cpre_out.at[pl.ds(c * NPAD + base, ROWS_PER_SUB)])


def _tc2_body(acc_ref, cpre_ref, hp_ref, dis_ref, b1_ref, w2_ref, b2_ref,
              out_ref):
    dis = dis_ref[...]
    sacc = acc_ref[0] + acc_ref[1] + hp_ref[...]
    out1 = sacc[0:N, :] * dis[0:N, None] + b1_ref[...][None, :]
    r = jnp.maximum(out1, 0.0)
    cpre = cpre_ref[0] + cpre_ref[1] + dis
    cw = dis * cpre
    v = jnp.sum(r * cw[0:N, None], axis=0)
    out = jnp.sum(w2_ref[...] * v[:, None], axis=0) * (1.0 / N) + b2_ref[...]
    out_ref[...] = out


def _tc2(acc_part, cpre_part, hp, dis, b1, w2, b2):
    return pl.pallas_call(
        _tc2_body,
        out_shape=jax.ShapeDtypeStruct((HID,), jnp.float32),
    )(acc_part, cpre_part, hp, dis, b1, w2, b2)


def kernel(neigborhood_state, edges, W1, b1, W2, b2):
    src = edges[0].astype(jnp.int32)
    dst = edges[1].astype(jnp.int32)
    pad = N + (jnp.arange(EPAD - E, dtype=jnp.int32) % (NPAD - N))
    src_d = jnp.concatenate([src, pad]).reshape(16, KD, W)
    dst_d = jnp.concatenate([dst, pad]).reshape(16, KD, W)

    h = _tc1(neigborhood_state, W1)
    acc_flat, cpre_flat, hp, dis = _main_kernel(dst_d, src_d, h)
    acc_part = acc_flat.reshape(2, NPAD, HID)
    cpre_part = cpre_flat.reshape(2, NPAD)
    return _tc2(acc_part, cpre_part, hp, dis, b1, W2, b2)


# speedup vs baseline: 1.3749x; 1.3749x over previous
import functools

import jax
import jax.numpy as jnp
from jax import lax
from jax.experimental import pallas as pl
from jax.experimental.pallas import tpu as pltpu
from jax.experimental.pallas import tpu_sc as plsc

N = 10000
E = 320000
HID = 16
NPAD = 10240
NW = 32
W = 128
K = 80
EPAD = NW * K * W
GROUP = 8
NG = K // GROUP
KD = 2 * K
NGD = KD // GROUP
ROWS_PER_SUB = NPAD // 16

_mesh = plsc.VectorSubcoreMesh(core_axis_name="c", subcore_axis_name="s")


def _newton_rsqrt(x):
    i = lax.bitcast_convert_type(x, jnp.int32)
    i = jnp.int32(0x5F3759DF) - lax.shift_right_arithmetic(i, 1)
    y = lax.bitcast_convert_type(i, jnp.float32)
    for _ in range(4):
        y = y * (1.5 - 0.5 * x * y * y)
    return y


def _tc1_body(x_ref, w1_ref, h_ref):
    w1 = w1_ref[...]
    wt = jnp.concatenate([w1] * 8, axis=0)
    wt = jnp.concatenate([wt] * 8, axis=1)
    prow = lax.broadcasted_iota(jnp.int32, (8 * 128, 8 * HID), 0) // 128
    qcol = lax.broadcasted_iota(jnp.int32, (8 * 128, 8 * HID), 1) // HID
    w2b = jnp.where(prow == qcol, wt, 0.0)
    x2 = x_ref[...].reshape(N // 8, 8 * 128)
    h2 = jnp.dot(x2, w2b, preferred_element_type=jnp.float32)
    h_ref[0:N * HID] = h2.reshape(N * HID)
    h_ref[N * HID:NPAD * HID] = jnp.zeros((NPAD * HID - N * HID,), jnp.float32)


def _tc1(x, w1):
    return pl.pallas_call(
        _tc1_body,
        out_shape=jax.ShapeDtypeStruct((NPAD * HID,), jnp.float32),
    )(x, w1)


@functools.partial(
    pl.kernel,
    out_type=(
        jax.ShapeDtypeStruct((2 * NPAD, HID), jnp.float32),
        jax.ShapeDtypeStruct((2 * NPAD, HID), jnp.float32),
        jax.ShapeDtypeStruct((NPAD, HID), jnp.float32),
        jax.ShapeDtypeStruct((NPAD, HID), jnp.float32),
    ),
    mesh=_mesh,
    scratch_types=[
        pltpu.VMEM((KD, W), jnp.int32),
        pltpu.VMEM((KD, W), jnp.int32),
        pltpu.VMEM((W,), jnp.float32),
        pltpu.VMEM((2, GROUP * W), jnp.float32),
        pltpu.VMEM((2, GROUP * W, HID), jnp.float32),
        pltpu.VMEM((ROWS_PER_SUB, HID), jnp.float32),
        pltpu.VMEM((ROWS_PER_SUB,), jnp.float32),
        pltpu.VMEM_SHARED((NPAD, HID), jnp.float32),
        pltpu.VMEM_SHARED((NPAD,), jnp.float32),
        pltpu.VMEM_SHARED((NPAD,), jnp.float32),
        pltpu.SemaphoreType.DMA,
        pltpu.SemaphoreType.DMA,
        pltpu.SemaphoreType.DMA,
        pltpu.SemaphoreType.DMA,
    ],
    compiler_params=pltpu.CompilerParams(use_tc_tiling_on_sc=False),
)
def _main_kernel(edg_hbm, h_hbm,
                 acc_out, cpre_out, hp_out, dise_out,
                 dstd_v, srcd_v, ones_v, vals_v, rows_v,
                 stage_v, zbuf,
                 acc_sh, cpre_sh, dis_sh, sem_gr, sem_gv, sem_sr, sem_sv):
    c = lax.axis_index("c")
    s = lax.axis_index("s")
    base = s * ROWS_PER_SUB
    cK = c * K

    def zrow(i, carry):
        stage_v[i, :] = jnp.zeros((16,), jnp.float32)
        return carry

    lax.fori_loop(0, ROWS_PER_SUB, zrow, 0)

    def zsca(i, carry):
        zbuf[pl.ds(i * 16, 16)] = jnp.zeros((16,), jnp.float32)
        return carry

    lax.fori_loop(0, ROWS_PER_SUB // 16, zsca, 0)
    for o in range(W // 16):
        ones_v[pl.ds(o * 16, 16)] = jnp.ones((16,), jnp.float32)
    pltpu.sync_copy(stage_v, acc_sh.at[pl.ds(base, ROWS_PER_SUB)])
    pltpu.sync_copy(zbuf, cpre_sh.at[pl.ds(base, ROWS_PER_SUB)])
    pltpu.sync_copy(zbuf, dis_sh.at[pl.ds(base, ROWS_PER_SUB)])
    pltpu.sync_copy(edg_hbm.at[1, s], dstd_v)
    pltpu.sync_copy(edg_hbm.at[0, s], srcd_v)
    plsc.subcore_barrier()

    def dfire(g):
        for k in range(GROUP):
            pltpu.async_copy(ones_v, dis_sh.at[dstd_v.at[g * GROUP + k]],
                             sem_gr, add=True)

    def ddrain(g):
        for k in range(GROUP):
            pltpu.make_async_copy(ones_v,
                                  dis_sh.at[dstd_v.at[g * GROUP + k]],
                                  sem_gr).wait()

    dfire(0)

    def dbody(g, carry):
        @pl.when(g + 1 < NGD)
        def _():
            dfire(g + 1)

        ddrain(g)
        return carry

    lax.fori_loop(0, NGD, dbody, 0)
    plsc.subcore_barrier()

    pltpu.sync_copy(dis_sh.at[pl.ds(base, ROWS_PER_SUB)], zbuf)
    pltpu.sync_copy(h_hbm.at[pl.ds(base, ROWS_PER_SUB)], stage_v)

    def dis_vec(i, carry):
        d = zbuf[pl.ds(i * 16, 16)] + 1.0
        zbuf[pl.ds(i * 16, 16)] = _newton_rsqrt(d)
        return carry

    lax.fori_loop(0, ROWS_PER_SUB // 16, dis_vec, 0)

    def scale_blk(i, carry):
        dv = zbuf[pl.ds(i * 16, 16)]
        for k in range(16):
            r = i * 16 + k
            stage_v[r, :] = stage_v[r, :] * dv[k]
            rows_v[0, r, :] = jnp.full((16,), 1.0, jnp.float32) * dv[k]
        return carry

    lax.fori_loop(0, ROWS_PER_SUB // 16, scale_blk, 0)
    pltpu.sync_copy(zbuf, dis_sh.at[pl.ds(base, ROWS_PER_SUB)])
    pltpu.sync_copy(stage_v, hp_out.at[pl.ds(base, ROWS_PER_SUB)])
    pltpu.sync_copy(rows_v.at[0, pl.ds(0, ROWS_PER_SUB)],
                    dise_out.at[pl.ds(base, ROWS_PER_SUB)])
    plsc.subcore_barrier()

    def fire_gathers(g, b):
        for k in range(GROUP):
            j = cK + g * GROUP + k
            pltpu.async_copy(hp_out.at[srcd_v.at[j]],
                             rows_v.at[b, pl.ds(k * W, W)], sem_gr)
            pltpu.async_copy(dis_sh.at[dstd_v.at[j]],
                             vals_v.at[b, pl.ds(k * W, W)], sem_gv)

    def drain_gathers(g, b):
        for k in range(GROUP):
            j = cK + g * GROUP + k
            pltpu.make_async_copy(hp_out.at[srcd_v.at[j]],
                                  rows_v.at[b, pl.ds(k * W, W)], sem_gr).wait()
            pltpu.make_async_copy(dis_sh.at[dstd_v.at[j]],
                                  vals_v.at[b, pl.ds(k * W, W)], sem_gv).wait()

    def fire_scatters(g, b):
        for k in range(GROUP):
            j = cK + g * GROUP + k
            pltpu.async_copy(rows_v.at[b, pl.ds(k * W, W)],
                             acc_sh.at[dstd_v.at[j]], sem_sr, add=True)
            pltpu.async_copy(vals_v.at[b, pl.ds(k * W, W)],
                             cpre_sh.at[srcd_v.at[j]], sem_sv, add=True)

    def drain_scatters(g, b):
        for k in range(GROUP):
            j = cK + g * GROUP + k
            pltpu.make_async_copy(rows_v.at[b, pl.ds(k * W, W)],
                                  acc_sh.at[dstd_v.at[j]], sem_sr).wait()
            pltpu.make_async_copy(vals_v.at[b, pl.ds(k * W, W)],
                                  cpre_sh.at[srcd_v.at[j]], sem_sv).wait()

    fire_gathers(0, 0)

    def body(g, carry):
        b = lax.rem(g, 2)
        drain_gathers(g, b)

        @pl.when(g + 1 < NG)
        def _():
            fire_gathers(g + 1, 1 - b)

        fire_scatters(g, b)
        drain_scatters(g, b)
        return carry

    lax.fori_loop(0, NG, body, 0)
    plsc.subcore_barrier()

    pltpu.sync_copy(acc_sh.at[pl.ds(base, ROWS_PER_SUB)], stage_v)
    pltpu.sync_copy(stage_v, acc_out.at[pl.ds(c * NPAD + base, ROWS_PER_SUB)])
    pltpu.sync_copy(cpre_sh.at[pl.ds(base, ROWS_PER_SUB)], zbuf)

    def cexp_blk(i, carry):
        dv = zbuf[pl.ds(i * 16, 16)]
        for k in range(16):
            r = i * 16 + k
            stage_v[r, :] = jnp.full((16,), 1.0, jnp.float32) * dv[k]
        return carry

    lax.fori_loop(0, ROWS_PER_SUB // 16, cexp_blk, 0)
    pltpu.sync_copy(stage_v, cpre_out.at[pl.ds(c * NPAD + base, ROWS_PER_SUB)])


_NR = N * HID // 128
_NRP = NPAD * HID // 128


def _tc2_body(accf_ref, cpef_ref, hpf_ref, disef_ref, b1_ref, w2_ref, b2_ref,
              out_ref):
    av = accf_ref[...].reshape(2 * _NRP, 128)
    cv = cpef_ref[...].reshape(2 * _NRP, 128)
    hv = hpf_ref[...].reshape(_NRP, 128)
    dv = disef_ref[...].reshape(_NRP, 128)
    b1t = jnp.concatenate([b1_ref[...]] * 8)
    f = av[0:_NRP] + av[_NRP:2 * _NRP] + hv
    r = jnp.maximum(f * dv + b1t[None, :], 0.0)
    ce = dv * (cv[0:_NRP] + cv[_NRP:2 * _NRP] + dv)
    u = r * ce
    rowid = lax.broadcasted_iota(jnp.int32, (_NRP, 128), 0)
    u = jnp.where(rowid < _NR, u, 0.0)
    v128 = jnp.sum(u, axis=0)
    v16 = v128[0:16]
    for i in range(1, 8):
        v16 = v16 + v128[16 * i:16 * (i + 1)]
    out = jnp.sum(w2_ref[...] * v16[:, None], axis=0) * (1.0 / N) + b2_ref[...]
    out_ref[...] = out


def _tc2(accf, cpef, hpf, disef, b1, w2, b2):
    return pl.pallas_call(
        _tc2_body,
        out_shape=jax.ShapeDtypeStruct((HID,), jnp.float32),
    )(accf, cpef, hpf, disef, b1, w2, b2)


def kernel(neigborhood_state, edges, W1, b1, W2, b2):
    pad = N + (jnp.arange(EPAD - E, dtype=jnp.int32) % (NPAD - N))
    epad = jnp.concatenate(
        [edges.astype(jnp.int32), jnp.stack([pad, pad])], axis=1
    ).reshape(2, 16, KD, W)

    h_flat = _tc1(neigborhood_state, W1)
    h2d = h_flat.reshape(NPAD, HID)
    acc, cpre_e, hp, dis_e = _main_kernel(epad, h2d)
    return _tc2(acc.reshape(2 * NPAD * HID), cpre_e.reshape(2 * NPAD * HID),
                hp.reshape(NPAD * HID), dis_e.reshape(NPAD * HID),
                b1, W2, b2)


# speedup vs baseline: 1.4384x; 1.0462x over previous
import functools

import jax
import jax.numpy as jnp
from jax import lax
from jax.experimental import pallas as pl
from jax.experimental.pallas import tpu as pltpu
from jax.experimental.pallas import tpu_sc as plsc

N = 10000
E = 320000
HID = 16
NPAD = 10240
NW = 32
W = 128
NWIN = E // W
K = 80
KD2 = 158
RING = 16
LAG = 8
ROWS_PER_SUB = NPAD // 16

_mesh = plsc.VectorSubcoreMesh(core_axis_name="c", subcore_axis_name="s")


def _newton_rsqrt(x):
    i = lax.bitcast_convert_type(x, jnp.int32)
    i = jnp.int32(0x5F3759DF) - lax.shift_right_arithmetic(i, 1)
    y = lax.bitcast_convert_type(i, jnp.float32)
    for _ in range(4):
        y = y * (1.5 - 0.5 * x * y * y)
    return y


def _tc1_body(x_ref, w1_ref, h_ref):
    w1 = w1_ref[...]
    wt = jnp.concatenate([w1] * 8, axis=0)
    wt = jnp.concatenate([wt] * 8, axis=1)
    prow = lax.broadcasted_iota(jnp.int32, (8 * 128, 8 * HID), 0) // 128
    qcol = lax.broadcasted_iota(jnp.int32, (8 * 128, 8 * HID), 1) // HID
    w2b = jnp.where(prow == qcol, wt, 0.0)
    x2 = x_ref[...].reshape(N // 8, 8 * 128)
    h2 = jnp.dot(x2, w2b, preferred_element_type=jnp.float32)
    h_ref[0:N * HID] = h2.reshape(N * HID)
    h_ref[N * HID:NPAD * HID] = jnp.zeros((NPAD * HID - N * HID,), jnp.float32)


def _tc1(x, w1):
    return pl.pallas_call(
        _tc1_body,
        out_shape=jax.ShapeDtypeStruct((NPAD * HID,), jnp.float32),
    )(x, w1)


@functools.partial(
    pl.kernel,
    out_type=(
        jax.ShapeDtypeStruct((2 * NPAD, HID), jnp.float32),
        jax.ShapeDtypeStruct((2 * NPAD, HID), jnp.float32),
        jax.ShapeDtypeStruct((NPAD, HID), jnp.float32),
        jax.ShapeDtypeStruct((NPAD, HID), jnp.float32),
    ),
    mesh=_mesh,
    scratch_types=[
        pltpu.VMEM((KD2 + 1, W), jnp.int32),
        pltpu.VMEM((K + 1, W), jnp.int32),
        pltpu.VMEM((K + 1, W), jnp.int32),
        pltpu.VMEM((W,), jnp.float32),
        pltpu.VMEM((RING, W), jnp.float32),
        pltpu.VMEM((RING, W, HID), jnp.float32),
        pltpu.VMEM((ROWS_PER_SUB, HID), jnp.float32),
        pltpu.VMEM((ROWS_PER_SUB,), jnp.float32),
        pltpu.VMEM_SHARED((NPAD, HID), jnp.float32),
        pltpu.VMEM_SHARED((NPAD,), jnp.float32),
        pltpu.VMEM_SHARED((NPAD,), jnp.float32),
        pltpu.SemaphoreType.DMA,
        pltpu.SemaphoreType.DMA,
        pltpu.SemaphoreType.DMA,
        pltpu.SemaphoreType.DMA,
    ],
    compiler_params=pltpu.CompilerParams(use_tc_tiling_on_sc=False),
)
def _main_kernel(edg_hbm, pad_hbm, h_hbm,
                 acc_out, cpre_out, hp_out, dise_out,
                 dsta_v, srcc_v, dstc_v, ones_v, vals_v, rows_v,
                 stage_v, zbuf,
                 acc_sh, cpre_sh, dis_sh, sem_gr, sem_gv, sem_sr, sem_sv):
    c = lax.axis_index("c")
    s = lax.axis_index("s")
    base = s * ROWS_PER_SUB

    def zrow(i, carry):
        stage_v[i, :] = jnp.zeros((16,), jnp.float32)
        return carry

    lax.fori_loop(0, ROWS_PER_SUB, zrow, 0)

    def zsca(i, carry):
        zbuf[pl.ds(i * 16, 16)] = jnp.zeros((16,), jnp.float32)
        return carry

    lax.fori_loop(0, ROWS_PER_SUB // 16, zsca, 0)
    for o in range(W // 16):
        ones_v[pl.ds(o * 16, 16)] = jnp.ones((16,), jnp.float32)
    pltpu.sync_copy(stage_v, acc_sh.at[pl.ds(base, ROWS_PER_SUB)])
    pltpu.sync_copy(zbuf, cpre_sh.at[pl.ds(base, ROWS_PER_SUB)])
    pltpu.sync_copy(zbuf, dis_sh.at[pl.ds(base, ROWS_PER_SUB)])

    sa = s * NWIN // 16
    na = (s + 1) * NWIN // 16 - sa
    pltpu.sync_copy(edg_hbm.at[1, pl.ds(sa, KD2 - 1)],
                    dsta_v.at[pl.ds(0, KD2 - 1)])
    pltpu.sync_copy(pad_hbm.at[1], dsta_v.at[pl.ds(na, 2)])

    w = 2 * s + c
    sc_ = w * NWIN // NW
    nc = (w + 1) * NWIN // NW - sc_
    pltpu.sync_copy(edg_hbm.at[0, pl.ds(sc_, K - 1)],
                    srcc_v.at[pl.ds(0, K - 1)])
    pltpu.sync_copy(pad_hbm.at[0], srcc_v.at[pl.ds(nc, 2)])
    pltpu.sync_copy(edg_hbm.at[1, pl.ds(sc_, K - 1)],
                    dstc_v.at[pl.ds(0, K - 1)])
    pltpu.sync_copy(pad_hbm.at[1], dstc_v.at[pl.ds(nc, 2)])
    plsc.subcore_barrier()

    def dfire(j):
        pltpu.async_copy(ones_v, dis_sh.at[dsta_v.at[j]], sem_gr, add=True)

    def ddrain(j):
        pltpu.make_async_copy(ones_v, dis_sh.at[dsta_v.at[j]], sem_gr).wait()

    def dbody(j, carry):
        @pl.when(j < KD2)
        def _():
            dfire(j)

        @pl.when(j >= RING)
        def _():
            ddrain(j - RING)

        return carry

    lax.fori_loop(0, KD2 + RING, dbody, 0)
    plsc.subcore_barrier()

    pltpu.sync_copy(dis_sh.at[pl.ds(base, ROWS_PER_SUB)], zbuf)
    pltpu.sync_copy(h_hbm.at[pl.ds(base, ROWS_PER_SUB)], stage_v)

    def dis_vec(i, carry):
        d = zbuf[pl.ds(i * 16, 16)] + 1.0
        zbuf[pl.ds(i * 16, 16)] = _newton_rsqrt(d)
        return carry

    lax.fori_loop(0, ROWS_PER_SUB // 16, dis_vec, 0)

    def scale_blk(i, carry):
        dv = zbuf[pl.ds(i * 16, 16)]
        for k in range(16):
            r = i * 16 + k
            stage_v[r, :] = stage_v[r, :] * dv[k]
        return carry

    lax.fori_loop(0, ROWS_PER_SUB // 16, scale_blk, 0)
    pltpu.sync_copy(zbuf, dis_sh.at[pl.ds(base, ROWS_PER_SUB)])
    pltpu.sync_copy(stage_v, hp_out.at[pl.ds(base, ROWS_PER_SUB)])

    def dexp_blk(i, carry):
        dv = zbuf[pl.ds(i * 16, 16)]
        for k in range(16):
            r = i * 16 + k
            stage_v[r, :] = jnp.full((16,), 1.0, jnp.float32) * dv[k]
        return carry

    lax.fori_loop(0, ROWS_PER_SUB // 16, dexp_blk, 0)
    pltpu.sync_copy(stage_v, dise_out.at[pl.ds(base, ROWS_PER_SUB)])
    plsc.subcore_barrier()

    def fire_g(j):
        slot = lax.rem(j, RING)
        pltpu.async_copy(hp_out.at[srcc_v.at[j]], rows_v.at[slot], sem_gr)
        pltpu.async_copy(dis_sh.at[dstc_v.at[j]], vals_v.at[slot], sem_gv)

    def drain_g(j):
        slot = lax.rem(j, RING)
        pltpu.make_async_copy(hp_out.at[srcc_v.at[j]],
                              rows_v.at[slot], sem_gr).wait()
        pltpu.make_async_copy(dis_sh.at[dstc_v.at[j]],
                              vals_v.at[slot], sem_gv).wait()

    def fire_s(j):
        slot = lax.rem(j, RING)
        pltpu.async_copy(rows_v.at[slot], acc_sh.at[dstc_v.at[j]],
                         sem_sr, add=True)
        pltpu.async_copy(vals_v.at[slot], cpre_sh.at[srcc_v.at[j]],
                         sem_sv, add=True)

    def drain_s(j):
        slot = lax.rem(j, RING)
        pltpu.make_async_copy(rows_v.at[slot],
                              acc_sh.at[dstc_v.at[j]], sem_sr).wait()
        pltpu.make_async_copy(vals_v.at[slot],
                              cpre_sh.at[srcc_v.at[j]], sem_sv).wait()

    def cbody(j, carry):
        @pl.when(j < K)
        def _():
            @pl.when(j >= RING)
            def _():
                drain_s(j - RING)

            fire_g(j)

        @pl.when(j >= LAG)
        def _():
            drain_g(j - LAG)
            fire_s(j - LAG)

        return carry

    lax.fori_loop(0, K + LAG, cbody, 0)
    for t in range(RING):
        drain_s(K - RING + t)
    plsc.subcore_barrier()

    pltpu.sync_copy(acc_sh.at[pl.ds(base, ROWS_PER_SUB)], stage_v)
    pltpu.sync_copy(stage_v, acc_out.at[pl.ds(c * NPAD + base, ROWS_PER_SUB)])
    pltpu.sync_copy(cpre_sh.at[pl.ds(base, ROWS_PER_SUB)], zbuf)

    def cexp_blk(i, carry):
        dv = zbuf[pl.ds(i * 16, 16)]
        for k in range(16):
            r = i * 16 + k
            stage_v[r, :] = jnp.full((16,), 1.0, jnp.float32) * dv[k]
        return carry

    lax.fori_loop(0, ROWS_PER_SUB // 16, cexp_blk, 0)
    pltpu.sync_copy(stage_v, cpre_out.at[pl.ds(c * NPAD + base, ROWS_PER_SUB)])


_NR = N * HID // 128
_NRP = NPAD * HID // 128


def _tc2_body(accf_ref, cpef_ref, hpf_ref, disef_ref, b1_ref, w2_ref, b2_ref,
              out_ref):
    av = accf_ref[...].reshape(2 * _NRP, 128)
    cv = cpef_ref[...].reshape(2 * _NRP, 128)
    hv = hpf_ref[...].reshape(_NRP, 128)
    dv = disef_ref[...].reshape(_NRP, 128)
    b1t = jnp.concatenate([b1_ref[...]] * 8)
    f = av[0:_NRP] + av[_NRP:2 * _NRP] + hv
    r = jnp.maximum(f * dv + b1t[None, :], 0.0)
    ce = dv * (cv[0:_NRP] + cv[_NRP:2 * _NRP] + dv)
    u = r * ce
    rowid = lax.broadcasted_iota(jnp.int32, (_NRP, 128), 0)
    u = jnp.where(rowid < _NR, u, 0.0)
    v128 = jnp.sum(u, axis=0)
    v16 = v128[0:16]
    for i in range(1, 8):
        v16 = v16 + v128[16 * i:16 * (i + 1)]
    out = jnp.sum(w2_ref[...] * v16[:, None], axis=0) * (1.0 / N) + b2_ref[...]
    out_ref[...] = out


def _tc2(accf, cpef, hpf, disef, b1, w2, b2):
    return pl.pallas_call(
        _tc2_body,
        out_shape=jax.ShapeDtypeStruct((HID,), jnp.float32),
    )(accf, cpef, hpf, disef, b1, w2, b2)


def kernel(neigborhood_state, edges, W1, b1, W2, b2):
    edg = edges.astype(jnp.int32).reshape(2, NWIN, W)
    padrow = N + (jnp.arange(2 * W, dtype=jnp.int32) % (NPAD - N))
    padblk = jnp.stack([padrow, padrow]).reshape(2, 2, W)

    h_flat = _tc1(neigborhood_state, W1)
    h2d = h_flat.reshape(NPAD, HID)
    acc, cpre_e, hp, dis_e = _main_kernel(edg, padblk, h2d)
    return _tc2(acc.reshape(2 * NPAD * HID), cpre_e.reshape(2 * NPAD * HID),
                hp.reshape(NPAD * HID), dis_e.reshape(NPAD * HID),
                b1, W2, b2)


# speedup vs baseline: 1.5745x; 1.0946x over previous
import functools

import jax
import jax.numpy as jnp
from jax import lax
from jax.experimental import pallas as pl
from jax.experimental.pallas import tpu as pltpu
from jax.experimental.pallas import tpu_sc as plsc

N = 10000
E = 320000
HID = 16
NPAD = 10240
NW = 32
W = 128
NWIN = E // W
K = 80
KD2 = 158
RING = 16
LAG = 8
ROWS_PER_SUB = NPAD // 16

_mesh = plsc.VectorSubcoreMesh(core_axis_name="c", subcore_axis_name="s")


def _newton_rsqrt(x):
    i = lax.bitcast_convert_type(x, jnp.int32)
    i = jnp.int32(0x5F3759DF) - lax.shift_right_arithmetic(i, 1)
    y = lax.bitcast_convert_type(i, jnp.float32)
    for _ in range(4):
        y = y * (1.5 - 0.5 * x * y * y)
    return y


def _tc1_body(x_ref, w1_ref, h_ref):
    w1 = w1_ref[...]
    wt = jnp.concatenate([w1] * 8, axis=0)
    wt = jnp.concatenate([wt] * 8, axis=1)
    prow = lax.broadcasted_iota(jnp.int32, (8 * 128, 8 * HID), 0) // 128
    qcol = lax.broadcasted_iota(jnp.int32, (8 * 128, 8 * HID), 1) // HID
    w2b = jnp.where(prow == qcol, wt, 0.0)
    x2 = x_ref[...].reshape(N // 8, 8 * 128)
    h2 = jnp.dot(x2, w2b, preferred_element_type=jnp.float32)
    h_ref[0:N * HID] = h2.reshape(N * HID)
    h_ref[N * HID:NPAD * HID] = jnp.zeros((NPAD * HID - N * HID,), jnp.float32)


def _tc1(x, w1):
    return pl.pallas_call(
        _tc1_body,
        out_shape=jax.ShapeDtypeStruct((NPAD * HID,), jnp.float32),
    )(x, w1)


@functools.partial(
    pl.kernel,
    out_type=(
        jax.ShapeDtypeStruct((2 * NPAD, HID), jnp.float32),
        jax.ShapeDtypeStruct((2 * NPAD, HID), jnp.float32),
        jax.ShapeDtypeStruct((NPAD, HID), jnp.float32),
        jax.ShapeDtypeStruct((NPAD, HID), jnp.float32),
    ),
    mesh=_mesh,
    scratch_types=[
        pltpu.VMEM((KD2 + 1, 2, W), jnp.int32),
        pltpu.VMEM((W,), jnp.float32),
        pltpu.VMEM((RING, W), jnp.float32),
        pltpu.VMEM((RING, W, HID), jnp.float32),
        pltpu.VMEM((ROWS_PER_SUB, HID), jnp.float32),
        pltpu.VMEM((ROWS_PER_SUB,), jnp.float32),
        pltpu.VMEM_SHARED((NPAD, HID), jnp.float32),
        pltpu.VMEM_SHARED((NPAD,), jnp.float32),
        pltpu.VMEM_SHARED((NPAD,), jnp.float32),
        pltpu.SemaphoreType.DMA,
        pltpu.SemaphoreType.DMA,
        pltpu.SemaphoreType.DMA,
        pltpu.SemaphoreType.DMA,
    ],
    compiler_params=pltpu.CompilerParams(use_tc_tiling_on_sc=False),
)
def _main_kernel(edg_hbm, pad_hbm, h_hbm,
                 acc_out, cpre_out, hp_out, dise_out,
                 ea_v, ones_v, vals_v, rows_v,
                 stage_v, zbuf,
                 acc_sh, cpre_sh, dis_sh, sem_gr, sem_gv, sem_sr, sem_sv):
    c = lax.axis_index("c")
    s = lax.axis_index("s")
    base = s * ROWS_PER_SUB

    def zrow(i, carry):
        stage_v[i, :] = jnp.zeros((16,), jnp.float32)
        return carry

    lax.fori_loop(0, ROWS_PER_SUB, zrow, 0)

    def zsca(i, carry):
        zbuf[pl.ds(i * 16, 16)] = jnp.zeros((16,), jnp.float32)
        return carry

    lax.fori_loop(0, ROWS_PER_SUB // 16, zsca, 0)
    for o in range(W // 16):
        ones_v[pl.ds(o * 16, 16)] = jnp.ones((16,), jnp.float32)
    pltpu.sync_copy(stage_v, acc_sh.at[pl.ds(base, ROWS_PER_SUB)])
    pltpu.sync_copy(zbuf, cpre_sh.at[pl.ds(base, ROWS_PER_SUB)])
    pltpu.sync_copy(zbuf, dis_sh.at[pl.ds(base, ROWS_PER_SUB)])

    sa = s * NWIN // 16
    na = (s + 1) * NWIN // 16 - sa
    pltpu.sync_copy(edg_hbm.at[pl.ds(sa, KD2 - 1)],
                    ea_v.at[pl.ds(0, KD2 - 1)])
    pltpu.sync_copy(pad_hbm, ea_v.at[pl.ds(na, 2)])

    w = 2 * s + c
    sc_ = w * NWIN // NW
    o0 = sc_ - sa
    nc = (w + 1) * NWIN // NW - sc_
    plsc.subcore_barrier()

    def dfire(j):
        pltpu.async_copy(ones_v, dis_sh.at[ea_v.at[j, 1]], sem_gr, add=True)

    def ddrain(j):
        pltpu.make_async_copy(ones_v, dis_sh.at[ea_v.at[j, 1]], sem_gr).wait()

    def dbody(j, carry):
        @pl.when(j < KD2)
        def _():
            dfire(j)

        @pl.when(j >= RING)
        def _():
            ddrain(j - RING)

        return carry

    lax.fori_loop(0, KD2 + RING, dbody, 0)
    plsc.subcore_barrier()
    pltpu.sync_copy(pad_hbm, ea_v.at[pl.ds(o0 + nc, 2)])

    pltpu.sync_copy(dis_sh.at[pl.ds(base, ROWS_PER_SUB)], zbuf)
    pltpu.sync_copy(h_hbm.at[pl.ds(base, ROWS_PER_SUB)], stage_v)

    def dis_vec(i, carry):
        d = zbuf[pl.ds(i * 16, 16)] + 1.0
        zbuf[pl.ds(i * 16, 16)] = _newton_rsqrt(d)
        return carry

    lax.fori_loop(0, ROWS_PER_SUB // 16, dis_vec, 0)

    def scale_blk(i, carry):
        dv = zbuf[pl.ds(i * 16, 16)]
        for k in range(16):
            r = i * 16 + k
            stage_v[r, :] = stage_v[r, :] * dv[k]
        return carry

    lax.fori_loop(0, ROWS_PER_SUB // 16, scale_blk, 0)
    pltpu.sync_copy(zbuf, dis_sh.at[pl.ds(base, ROWS_PER_SUB)])
    pltpu.sync_copy(stage_v, hp_out.at[pl.ds(base, ROWS_PER_SUB)])

    def dexp_blk(i, carry):
        dv = zbuf[pl.ds(i * 16, 16)]
        for k in range(16):
            r = i * 16 + k
            stage_v[r, :] = jnp.full((16,), 1.0, jnp.float32) * dv[k]
        return carry

    lax.fori_loop(0, ROWS_PER_SUB // 16, dexp_blk, 0)
    pltpu.sync_copy(stage_v, dise_out.at[pl.ds(base, ROWS_PER_SUB)])
    plsc.subcore_barrier()

    def fire_g(j):
        slot = lax.rem(j, RING)
        jj = o0 + j
        pltpu.async_copy(hp_out.at[ea_v.at[jj, 0]], rows_v.at[slot], sem_gr)
        pltpu.async_copy(dis_sh.at[ea_v.at[jj, 1]], vals_v.at[slot], sem_gv)

    def drain_g(j):
        slot = lax.rem(j, RING)
        jj = o0 + j
        pltpu.make_async_copy(hp_out.at[ea_v.at[jj, 0]],
                              rows_v.at[slot], sem_gr).wait()
        pltpu.make_async_copy(dis_sh.at[ea_v.at[jj, 1]],
                              vals_v.at[slot], sem_gv).wait()

    def fire_s(j):
        slot = lax.rem(j, RING)
        jj = o0 + j
        pltpu.async_copy(rows_v.at[slot], acc_sh.at[ea_v.at[jj, 1]],
                         sem_sr, add=True)
        pltpu.async_copy(vals_v.at[slot], cpre_sh.at[ea_v.at[jj, 0]],
                         sem_sv, add=True)

    def drain_s(j):
        slot = lax.rem(j, RING)
        jj = o0 + j
        pltpu.make_async_copy(rows_v.at[slot],
                              acc_sh.at[ea_v.at[jj, 1]], sem_sr).wait()
        pltpu.make_async_copy(vals_v.at[slot],
                              cpre_sh.at[ea_v.at[jj, 0]], sem_sv).wait()

    def cbody(j, carry):
        @pl.when(j < K)
        def _():
            @pl.when(j >= RING)
            def _():
                drain_s(j - RING)

            fire_g(j)

        @pl.when(j >= LAG)
        def _():
            drain_g(j - LAG)
            fire_s(j - LAG)

        return carry

    lax.fori_loop(0, K + LAG, cbody, 0)
    for t in range(RING):
        drain_s(K - RING + t)
    plsc.subcore_barrier()

    pltpu.sync_copy(acc_sh.at[pl.ds(base, ROWS_PER_SUB)], stage_v)
    pltpu.sync_copy(stage_v, acc_out.at[pl.ds(c * NPAD + base, ROWS_PER_SUB)])
    pltpu.sync_copy(cpre_sh.at[pl.ds(base, ROWS_PER_SUB)], zbuf)

    def cexp_blk(i, carry):
        dv = zbuf[pl.ds(i * 16, 16)]
        for k in range(16):
            r = i * 16 + k
            stage_v[r, :] = jnp.full((16,), 1.0, jnp.float32) * dv[k]
        return carry

    lax.fori_loop(0, ROWS_PER_SUB // 16, cexp_blk, 0)
    pltpu.sync_copy(stage_v, cpre_out.at[pl.ds(c * NPAD + base, ROWS_PER_SUB)])


_NR = N * HID // 128
_NRP = NPAD * HID // 128


def _tc2_body(accf_ref, cpef_ref, hpf_ref, disef_ref, b1_ref, w2_ref, b2_ref,
              out_ref):
    av = accf_ref[...].reshape(2 * _NRP, 128)
    cv = cpef_ref[...].reshape(2 * _NRP, 128)
    hv = hpf_ref[...].reshape(_NRP, 128)
    dv = disef_ref[...].reshape(_NRP, 128)
    b1t = jnp.concatenate([b1_ref[...]] * 8)
    f = av[0:_NRP] + av[_NRP:2 * _NRP] + hv
    r = jnp.maximum(f * dv + b1t[None, :], 0.0)
    ce = dv * (cv[0:_NRP] + cv[_NRP:2 * _NRP] + dv)
    u = r * ce
    rowid = lax.broadcasted_iota(jnp.int32, (_NRP, 128), 0)
    u = jnp.where(rowid < _NR, u, 0.0)
    v128 = jnp.sum(u, axis=0)
    v16 = v128[0:16]
    for i in range(1, 8):
        v16 = v16 + v128[16 * i:16 * (i + 1)]
    out = jnp.sum(w2_ref[...] * v16[:, None], axis=0) * (1.0 / N) + b2_ref[...]
    out_ref[...] = out


def _tc2(accf, cpef, hpf, disef, b1, w2, b2):
    return pl.pallas_call(
        _tc2_body,
        out_shape=jax.ShapeDtypeStruct((HID,), jnp.float32),
    )(accf, cpef, hpf, disef, b1, w2, b2)


def kernel(neigborhood_state, edges, W1, b1, W2, b2):
    edg = edges.astype(jnp.int32).reshape(2, NWIN, W).transpose(1, 0, 2)
    padrow = N + (jnp.arange(2 * W, dtype=jnp.int32) % (NPAD - N))
    padblk = jnp.broadcast_to(padrow.reshape(2, 1, W), (2, 2, W))

    h_flat = _tc1(neigborhood_state, W1)
    h2d = h_flat.reshape(NPAD, HID)
    acc, cpre_e, hp, dis_e = _main_kernel(edg, padblk, h2d)
    return _tc2(acc.reshape(2 * NPAD * HID), cpre_e.reshape(2 * NPAD * HID),
                hp.reshape(NPAD * HID), dis_e.reshape(NPAD * HID),
                b1, W2, b2)
